# baseline (speedup 1.0000x reference)
import functools

import jax
import jax.numpy as jnp
from jax import lax
from jax.experimental import pallas as pl
from jax.experimental.pallas import tpu as pltpu
from jax.experimental.pallas import tpu_sc as plsc


def _emb_call(n_rows, v_rows, d, idx3, table):
    info = plsc.get_sparse_core_info()
    nw = info.num_cores * info.num_subcores
    per_w = n_rows // nw
    ch = 128
    k = per_w // ch

    mesh = plsc.VectorSubcoreMesh(core_axis_name="c", subcore_axis_name="s")

    @functools.partial(
        pl.kernel,
        mesh=mesh,
        out_type=jax.ShapeDtypeStruct((n_rows, d), jnp.float32),
        scratch_types=[
            pltpu.VMEM((k, ch), jnp.int32),
            pltpu.VMEM((ch, d), jnp.float32),
            pltpu.SemaphoreType.DMA,
        ],
        compiler_params=pltpu.CompilerParams(use_tc_tiling_on_sc=False),
    )
    def emb(idx_hbm, table_hbm, out_hbm, idx_v, rows_v, sem):
        wid = lax.axis_index("s") * info.num_cores + lax.axis_index("c")
        base = wid * per_w
        pltpu.sync_copy(idx_hbm.at[wid], idx_v)

        def body(j, carry):
            pltpu.async_copy(table_hbm.at[idx_v.at[j]], rows_v, sem).wait()
            pltpu.sync_copy(rows_v, out_hbm.at[pl.ds(base + j * ch, ch)])
            return carry

        lax.fori_loop(0, k, body, 0)

    return emb(idx3, table)


def kernel(x, table):
    b, s = x.shape
    v_rows, d = table.shape
    n_rows = b * s
    idx3 = x.reshape(32, n_rows // (32 * 128), 128).astype(jnp.int32)
    out = _emb_call(n_rows, v_rows, d, idx3, table)
    return out.reshape(b, s, d)


# speedup vs baseline: 1.1180x; 1.1180x over previous
import functools

import jax
import jax.numpy as jnp
from jax import lax
from jax.experimental import pallas as pl
from jax.experimental.pallas import tpu as pltpu
from jax.experimental.pallas import tpu_sc as plsc

_NBUF = 4
_RING = 8


def _emb_call(n_rows, d, idx3, table):
    info = plsc.get_sparse_core_info()
    nw = info.num_cores * info.num_subcores
    per_w = n_rows // nw
    ch = 128
    k = per_w // ch
    n, r = _NBUF, _RING
    assert k > 2 * n and (k - 2 * n) % r == 0

    mesh = plsc.VectorSubcoreMesh(core_axis_name="c", subcore_axis_name="s")

    @functools.partial(
        pl.kernel,
        mesh=mesh,
        out_type=jax.ShapeDtypeStruct((n_rows, d), jnp.float32),
        scratch_types=[
            pltpu.VMEM((k, ch), jnp.int32),
            pltpu.VMEM((r, ch, d), jnp.float32),
            pltpu.SemaphoreType.DMA((r,)),
            pltpu.SemaphoreType.DMA((r,)),
        ],
        compiler_params=pltpu.CompilerParams(use_tc_tiling_on_sc=False),
    )
    def emb(idx_hbm, table_hbm, out_hbm, idx_v, rows_v, gsem, wsem):
        wid = lax.axis_index("s") * info.num_cores + lax.axis_index("c")
        base = wid * per_w
        pltpu.sync_copy(idx_hbm.at[wid], idx_v)

        def g_start(j, b):
            pltpu.async_copy(table_hbm.at[idx_v.at[j]], rows_v.at[b], gsem.at[b])

        def g_wait(j, b):
            pltpu.make_async_copy(
                table_hbm.at[idx_v.at[j]], rows_v.at[b], gsem.at[b]).wait()

        def w_start(j, b):
            pltpu.async_copy(
                rows_v.at[b], out_hbm.at[pl.ds(base + j * ch, ch)], wsem.at[b])

        def w_wait(b):
            pltpu.make_async_copy(
                rows_v.at[b], out_hbm.at[pl.ds(base, ch)], wsem.at[b]).wait()

        for b in range(n):
            g_start(b, b)
        for j in range(n):
            g_wait(j, j)
            w_start(j, j)
            g_start(j + n, j + n)
        def body(t, carry):
            g0 = n + r * t
            for b in range(r):
                j = g0 + b
                g_wait(j, (n + b) % r)
                w_start(j, (n + b) % r)
                w_wait(b)
                g_start(j + n, b)
            return carry

        lax.fori_loop(0, (k - 2 * n) // r, body, 0)
        for j in range(k - n, k):
            g_wait(j, j % r)
            w_start(j, j % r)
        for b in range(r):
            w_wait(b)

    return emb(idx3, table)


def kernel(x, table):
    b, s = x.shape
    _, d = table.shape
    n_rows = b * s
    idx3 = x.reshape(32, n_rows // (32 * 128), 128).astype(jnp.int32)
    out = _emb_call(n_rows, d, idx3, table)
    return out.reshape(b, s, d)
